# Initial kernel scaffold; baseline (speedup 1.0000x reference)
import jax
import jax.numpy as jnp
from jax.experimental import pallas as pl


def kernel(node_feats, edge_feats, neighbors, W, b):
    raise NotImplementedError("write your pallas kernel here")


# speedup vs baseline: 2.4259x; 2.4259x over previous
import functools

import jax
import jax.numpy as jnp
from jax import lax
from jax.experimental import pallas as pl
from jax.experimental.pallas import tpu as pltpu
from jax.experimental.pallas import tpu_sc as plsc

_NW = 32
_LANES = 16
_CH = 4


def _sc_gather_sum(node_feats, edge_feats, nidx_flat, eidx_flat, deg):
    n, d = node_feats.shape
    idx_per_chunk = _CH * deg
    nchunks = n // _CH
    base_cnt = nchunks // _NW
    rem = nchunks % _NW
    nvec = d // _LANES

    mesh = plsc.VectorSubcoreMesh(core_axis_name="c", subcore_axis_name="s")

    @functools.partial(
        pl.kernel,
        mesh=mesh,
        out_type=[jax.ShapeDtypeStruct((n, d), jnp.float32),
                  jax.ShapeDtypeStruct((n, d), jnp.float32)],
        scratch_types=[
            pltpu.VMEM((idx_per_chunk,), jnp.int32),
            pltpu.VMEM((idx_per_chunk,), jnp.int32),
            pltpu.VMEM((idx_per_chunk, d), jnp.float32),
            pltpu.VMEM((idx_per_chunk, d), jnp.float32),
            pltpu.VMEM((_CH, d), jnp.float32),
            pltpu.VMEM((_CH, d), jnp.float32),
            pltpu.SemaphoreType.DMA,
            pltpu.SemaphoreType.DMA,
        ],
    )
    def sc_kernel(node_hbm, edge_hbm, nidx_hbm, eidx_hbm, gn_hbm, ge_hbm,
                  nidx_v, eidx_v, nrows_v, erows_v, accn_v, acce_v,
                  sem_n, sem_e):
        wid = lax.axis_index("s") * 2 + lax.axis_index("c")
        cnt = base_cnt + jnp.where(wid < rem, 1, 0)
        start = wid * base_cnt + jnp.minimum(wid, rem)

        def chunk_body(k, carry):
            c = start + k
            node_base = c * _CH
            idx_base = c * idx_per_chunk
            pltpu.sync_copy(nidx_hbm.at[pl.ds(idx_base, idx_per_chunk)], nidx_v)
            pltpu.sync_copy(eidx_hbm.at[pl.ds(idx_base, idx_per_chunk)], eidx_v)
            cp_n = pltpu.async_copy(node_hbm.at[nidx_v], nrows_v, sem_n)
            cp_e = pltpu.async_copy(edge_hbm.at[eidx_v], erows_v, sem_e)
            cp_n.wait()
            cp_e.wait()
            for node in range(_CH):
                def row_body(r, accs, node=node):
                    row = node * deg + r
                    upd_n = tuple(accs[j] + nrows_v[row, pl.ds(_LANES * j, _LANES)]
                                  for j in range(nvec))
                    upd_e = tuple(accs[nvec + j] + erows_v[row, pl.ds(_LANES * j, _LANES)]
                                  for j in range(nvec))
                    return upd_n + upd_e
                zeros = tuple(jnp.zeros((_LANES,), jnp.float32) for _ in range(2 * nvec))
                accs = lax.fori_loop(0, deg, row_body, zeros)
                for j in range(nvec):
                    accn_v[node, pl.ds(_LANES * j, _LANES)] = accs[j]
                    acce_v[node, pl.ds(_LANES * j, _LANES)] = accs[nvec + j]
            pltpu.sync_copy(accn_v, gn_hbm.at[pl.ds(node_base, _CH)])
            pltpu.sync_copy(acce_v, ge_hbm.at[pl.ds(node_base, _CH)])
            return carry

        lax.fori_loop(0, cnt, chunk_body, 0)

    return sc_kernel(node_feats, edge_feats, nidx_flat, eidx_flat)


def _tc_finish(node_feats, gn, ge, w, b2, scale):
    n, d = node_feats.shape
    h = w.shape[0]
    br = 400
    grid = (n // br,)

    def body(x_ref, gn_ref, ge_ref, w_ref, b_ref, o_ref):
        xs = x_ref[...] + gn_ref[...]
        acc = lax.dot_general(xs, w_ref[...], (((1,), (1,)), ((), ())),
                              preferred_element_type=jnp.float32)
        o_ref[...] = acc + ge_ref[...] + scale * b_ref[...]

    return pl.pallas_call(
        body,
        grid=grid,
        in_specs=[
            pl.BlockSpec((br, d), lambda i: (i, 0)),
            pl.BlockSpec((br, h), lambda i: (i, 0)),
            pl.BlockSpec((br, h), lambda i: (i, 0)),
            pl.BlockSpec((h, d), lambda i: (0, 0)),
            pl.BlockSpec((1, h), lambda i: (0, 0)),
        ],
        out_specs=pl.BlockSpec((br, h), lambda i: (i, 0)),
        out_shape=jax.ShapeDtypeStruct((n, h), jnp.float32),
    )(node_feats, gn, ge, w, b2)


def kernel(node_feats, edge_feats, neighbors, W, b):
    deg = neighbors.shape[1]
    n_idx = neighbors[:, :, 0].reshape(-1)
    e_idx = neighbors[:, :, 1].reshape(-1)
    gn, ge = _sc_gather_sum(node_feats, edge_feats, n_idx, e_idx, deg)
    return _tc_finish(node_feats, gn, ge, W, b.reshape(1, -1),
                      float(deg + 1))
